# Initial kernel scaffold; baseline (speedup 1.0000x reference)
import functools

import jax
import jax.numpy as jnp
from jax.experimental import pallas as pl
from jax.experimental.pallas import tpu as pltpu

_TOP_K = 2
_TS = 1024
_TH = 1024


def _gate_kernel(x_ref, wg1_ref, bg1_ref, wg2_ref, bg2_ref, w_out, i_out):
    e = wg2_ref.shape[-1]
    xm = jnp.mean(x_ref[...], axis=1)
    gh = jnp.dot(xm, wg1_ref[...], preferred_element_type=jnp.float32,
                 precision=jax.lax.Precision.HIGHEST) + bg1_ref[...]
    gh = gh * jax.lax.logistic(gh)
    logits = jnp.dot(gh, wg2_ref[...], preferred_element_type=jnp.float32,
                     precision=jax.lax.Precision.HIGHEST) + bg2_ref[...]
    cols = jax.lax.broadcasted_iota(jnp.int32, logits.shape, 1)
    m1 = jnp.max(logits, axis=-1, keepdims=True)
    i1 = jnp.min(jnp.where(logits == m1, cols, e), axis=-1, keepdims=True)
    masked = jnp.where(cols == i1, -jnp.inf, logits)
    m2 = jnp.max(masked, axis=-1, keepdims=True)
    i2 = jnp.min(jnp.where(masked == m2, cols, e), axis=-1, keepdims=True)
    e2 = jnp.exp(m2 - m1)
    w1 = 1.0 / (1.0 + e2)
    w_out[...] = jnp.concatenate([w1, w1 * e2], axis=-1)
    i_out[...] = jnp.concatenate([i1, i2], axis=-1).astype(jnp.int32)


def _gate(x, Wg1, bg1, Wg2, bg2, *, top_k, interpret=False):
    b = x.shape[0]
    return pl.pallas_call(
        _gate_kernel,
        out_shape=(jax.ShapeDtypeStruct((b, top_k), jnp.float32),
                   jax.ShapeDtypeStruct((b, top_k), jnp.int32)),
        interpret=interpret,
    )(x, Wg1, bg1[None, :], Wg2, bg2[None, :])


def _moe_kernel(idx_ref, wts_ref, x_ref, w1_ref, b1_ref, w2_ref, b2_ref,
                out_ref, *, top_k):
    bk = pl.program_id(1)
    h = pl.program_id(2)
    w = wts_ref[bk]
    hmat = jnp.dot(x_ref[0], w1_ref[0], preferred_element_type=jnp.float32,
                   precision=jax.lax.Precision.HIGHEST) + b1_ref[...]
    hmat = hmat * jax.lax.logistic(hmat)
    part = jnp.dot(hmat, w2_ref[0], preferred_element_type=jnp.float32,
                   precision=jax.lax.Precision.HIGHEST)
    contrib = w * part
    first_of_block = jnp.logical_and(bk % top_k == 0, h == 0)

    @pl.when(first_of_block)
    def _():
        out_ref[0] = contrib + w * b2_ref[...]

    @pl.when(jnp.logical_and(h == 0, bk % top_k != 0))
    def _():
        out_ref[0] = out_ref[0] + (contrib + w * b2_ref[...])

    @pl.when(h != 0)
    def _():
        out_ref[0] = out_ref[0] + contrib


def _moe(x, W1, b1, W2, b2, idx_flat, wts_flat, *, ts, th, top_k,
         interpret=False):
    b, s, d = x.shape
    _, _, hdim = W1.shape
    grid = (s // ts, b * top_k, hdim // th)
    grid_spec = pltpu.PrefetchScalarGridSpec(
        num_scalar_prefetch=2,
        grid=grid,
        in_specs=[
            pl.BlockSpec((1, ts, d), lambda si, bk, hi, idx, wts: (bk // top_k, si, 0)),
            pl.BlockSpec((1, d, th), lambda si, bk, hi, idx, wts: (idx[bk], 0, hi)),
            pl.BlockSpec((1, th), lambda si, bk, hi, idx, wts: (idx[bk], hi)),
            pl.BlockSpec((1, th, d), lambda si, bk, hi, idx, wts: (idx[bk], hi, 0)),
            pl.BlockSpec((1, d), lambda si, bk, hi, idx, wts: (idx[bk], 0)),
        ],
        out_specs=pl.BlockSpec((1, ts, d),
                               lambda si, bk, hi, idx, wts: (bk // top_k, si, 0)),
    )
    return pl.pallas_call(
        functools.partial(_moe_kernel, top_k=top_k),
        grid_spec=grid_spec,
        out_shape=jax.ShapeDtypeStruct((b, s, d), jnp.float32),
        interpret=interpret,
    )(idx_flat, wts_flat, x, W1, b1, W2, b2)


def kernel(x, Wg1, bg1, Wg2, bg2, W1, b1, W2, b2):
    wts, idx = _gate(x, Wg1, bg1, Wg2, bg2, top_k=_TOP_K)
    out = _moe(x, W1, b1, W2, b2, idx.reshape(-1), wts.reshape(-1),
               ts=_TS, th=_TH, top_k=_TOP_K)
    return (out, (wts, idx))


# speedup vs baseline: 2.2090x; 2.2090x over previous
import functools

import jax
import jax.numpy as jnp
from jax.experimental import pallas as pl
from jax.experimental.pallas import tpu as pltpu

_TOP_K = 2
_TS = 2048
_TH = 512


def _gate_kernel(x_ref, wg1_ref, bg1_ref, wg2_ref, bg2_ref, w_out, i_out,
                 xbf_out):
    e = wg2_ref.shape[-1]
    xbf_out[...] = x_ref[...].astype(jnp.bfloat16)
    xm = jnp.mean(x_ref[...], axis=1)
    gh = jnp.dot(xm, wg1_ref[...], preferred_element_type=jnp.float32,
                 precision=jax.lax.Precision.HIGHEST) + bg1_ref[...]
    gh = gh * jax.lax.logistic(gh)
    logits = jnp.dot(gh, wg2_ref[...], preferred_element_type=jnp.float32,
                     precision=jax.lax.Precision.HIGHEST) + bg2_ref[...]
    cols = jax.lax.broadcasted_iota(jnp.int32, logits.shape, 1)
    m1 = jnp.max(logits, axis=-1, keepdims=True)
    i1 = jnp.min(jnp.where(logits == m1, cols, e), axis=-1, keepdims=True)
    masked = jnp.where(cols == i1, -jnp.inf, logits)
    m2 = jnp.max(masked, axis=-1, keepdims=True)
    i2 = jnp.min(jnp.where(masked == m2, cols, e), axis=-1, keepdims=True)
    e2 = jnp.exp(m2 - m1)
    w1 = 1.0 / (1.0 + e2)
    w_out[...] = jnp.concatenate([w1, w1 * e2], axis=-1)
    i_out[...] = jnp.concatenate([i1, i2], axis=-1).astype(jnp.int32)


def _gate(x, Wg1, bg1, Wg2, bg2, *, top_k, interpret=False):
    b, s, d = x.shape
    return pl.pallas_call(
        _gate_kernel,
        out_shape=(jax.ShapeDtypeStruct((b, top_k), jnp.float32),
                   jax.ShapeDtypeStruct((b, top_k), jnp.int32),
                   jax.ShapeDtypeStruct((b, s, d), jnp.bfloat16)),
        interpret=interpret,
    )(x, Wg1, bg1[None, :], Wg2, bg2[None, :])


def _moe_kernel(idx_ref, wts_ref, x_ref, w1_ref, b1_ref, w2_ref, b2_ref,
                out_ref, *, top_k):
    bk = pl.program_id(1)
    h = pl.program_id(2)
    w = wts_ref[bk]
    hmat = jnp.dot(x_ref[0], w1_ref[0].astype(jnp.bfloat16),
                   preferred_element_type=jnp.float32) + b1_ref[0]
    hmat = hmat * jax.lax.logistic(hmat)
    part = jnp.dot(hmat.astype(jnp.bfloat16), w2_ref[0].astype(jnp.bfloat16),
                   preferred_element_type=jnp.float32)
    contrib = w * part
    first_of_block = jnp.logical_and(bk % top_k == 0, h == 0)

    @pl.when(first_of_block)
    def _():
        out_ref[0] = contrib + w * b2_ref[0]

    @pl.when(jnp.logical_and(h == 0, bk % top_k != 0))
    def _():
        out_ref[0] = out_ref[0] + (contrib + w * b2_ref[0])

    @pl.when(h != 0)
    def _():
        out_ref[0] = out_ref[0] + contrib


def _moe(x_bf, W1, b1, W2, b2, idx_flat, wts_flat, *, ts, th, top_k,
         interpret=False):
    b, s, d = x_bf.shape
    _, _, hdim = W1.shape
    grid = (s // ts, b * top_k, hdim // th)
    grid_spec = pltpu.PrefetchScalarGridSpec(
        num_scalar_prefetch=2,
        grid=grid,
        in_specs=[
            pl.BlockSpec((1, ts, d), lambda si, bk, hi, idx, wts: (bk // top_k, si, 0)),
            pl.BlockSpec((1, d, th), lambda si, bk, hi, idx, wts: (idx[bk], 0, hi)),
            pl.BlockSpec((1, 1, th), lambda si, bk, hi, idx, wts: (idx[bk], 0, hi)),
            pl.BlockSpec((1, th, d), lambda si, bk, hi, idx, wts: (idx[bk], hi, 0)),
            pl.BlockSpec((1, 1, d), lambda si, bk, hi, idx, wts: (idx[bk], 0, 0)),
        ],
        out_specs=pl.BlockSpec((1, ts, d),
                               lambda si, bk, hi, idx, wts: (bk // top_k, si, 0)),
    )
    return pl.pallas_call(
        functools.partial(_moe_kernel, top_k=top_k),
        grid_spec=grid_spec,
        out_shape=jax.ShapeDtypeStruct((b, s, d), jnp.float32),
        interpret=interpret,
    )(idx_flat, wts_flat, x_bf, W1, b1[:, None, :], W2, b2[:, None, :])


def kernel(x, Wg1, bg1, Wg2, bg2, W1, b1, W2, b2):
    wts, idx, x_bf = _gate(x, Wg1, bg1, Wg2, bg2, top_k=_TOP_K)
    out = _moe(x_bf, W1, b1, W2, b2, idx.reshape(-1), wts.reshape(-1),
               ts=_TS, th=_TH, top_k=_TOP_K)
    return (out, (wts, idx))
